# baseline (speedup 1.0000x reference)
import functools

import jax
import jax.numpy as jnp
from jax import lax
from jax.experimental import pallas as pl
from jax.experimental.pallas import tpu as pltpu
from jax.experimental.pallas import tpu_sc as plsc

_NUM_CORES = 2
_NUM_SUBCORES = 16
_NUM_WORKERS = _NUM_CORES * _NUM_SUBCORES

_BLK = 128
_NBUF = 4


@functools.lru_cache(maxsize=None)
def _make_gather(n_b: int, n_s: int, d: int):
    assert d == 64 and n_b % _BLK == 0
    n_bh = n_b // _BLK
    n_blocks = n_s * n_bh
    blocks_per_w = n_blocks // _NUM_WORKERS
    assert blocks_per_w % _NBUF == 0 and blocks_per_w >= 2 * _NBUF

    mesh = plsc.VectorSubcoreMesh(
        core_axis_name="c", subcore_axis_name="s",
        num_cores=_NUM_CORES, num_subcores=_NUM_SUBCORES)

    @functools.partial(
        pl.kernel,
        out_type=jax.ShapeDtypeStruct((n_s, 8, n_bh, 8, _BLK), jnp.float32),
        mesh=mesh,
        scratch_types=[
            pltpu.VMEM((blocks_per_w, _BLK), jnp.int32),
            [pltpu.VMEM((_BLK, d), jnp.float32)] * _NBUF,
            [pltpu.VMEM((8, 8, _BLK), jnp.float32)] * _NBUF,
            [pltpu.SemaphoreType.DMA] * _NBUF,
            [pltpu.SemaphoreType.DMA] * _NBUF,
        ],
        compiler_params=pltpu.CompilerParams(
            use_tc_tiling_on_sc=False, needs_layout_passes=False),
    )
    def gather_kernel(table, idx_hbm, out5, idx_v, bufs, tbufs, gsems, ssems):
        wid = lax.axis_index("s") * _NUM_CORES + lax.axis_index("c")
        blk_base = wid * blocks_per_w
        pltpu.sync_copy(idx_hbm.at[pl.ds(blk_base, blocks_per_w)], idx_v)

        d16 = lax.iota(jnp.int32, 16)
        dh_vecs = [(d16 + 16 * k) >> 3 for k in range(4)]
        dl_vecs = [(d16 + 16 * k) & 7 for k in range(4)]

        def fire_gather(i, b):
            pltpu.async_copy(table.at[idx_v.at[i]], bufs[b], gsems[b])

        def wait_gather(b):
            pltpu.make_async_copy(
                table.at[idx_v.at[0]], bufs[b], gsems[b]).wait()

        def fire_store(i, b):
            blk = blk_base + i
            s = blk >> 5
            bh = blk & (n_bh - 1)
            pltpu.async_copy(tbufs[b], out5.at[s, :, bh], ssems[b])

        def wait_store(b):
            pltpu.make_async_copy(
                tbufs[b], out5.at[0, :, 0], ssems[b]).wait()

        def transpose_block(b):
            buf_r, tbuf_r = bufs[b], tbufs[b]

            @pl.loop(0, _BLK, unroll=4)
            def _(bl):
                blv = jnp.full((16,), bl, jnp.int32)
                for k in range(4):
                    v = buf_r[bl, pl.ds(16 * k, 16)]
                    plsc.store_scatter(
                        tbuf_r, [dh_vecs[k], dl_vecs[k], blv], v)

        def step(i, j, guard_store, guard_gather):
            if guard_store:
                wait_store(j)
            ahead = (j + 2) % _NBUF
            if guard_gather:

                @pl.when(i + 2 < blocks_per_w)
                def _():
                    fire_gather(i + 2, ahead)
            else:
                fire_gather(i + 2, ahead)
            wait_gather(j)
            transpose_block(j)
            fire_store(i, j)

        fire_gather(0, 0)
        fire_gather(1, 1)
        for j in range(_NBUF):
            step(j, j, guard_store=False, guard_gather=False)

        @pl.loop(1, blocks_per_w // _NBUF)
        def _(t):
            for j in range(_NBUF):
                step(_NBUF * t + j, j, guard_store=True, guard_gather=True)

        for j in range(_NBUF):
            wait_store(j)

    return gather_kernel


def kernel(inputs, embedding):
    b, s = inputs.shape
    v, d = embedding.shape
    idx = inputs.T.reshape(-1, _BLK).astype(jnp.int32)
    y = _make_gather(b, s, d)(embedding, idx)
    return y.transpose(2, 4, 0, 1, 3).reshape(b, s, d)


# speedup vs baseline: 2.3031x; 2.3031x over previous
import functools

import jax
import jax.numpy as jnp
from jax import lax
from jax.experimental import pallas as pl
from jax.experimental.pallas import tpu as pltpu
from jax.experimental.pallas import tpu_sc as plsc

_NUM_CORES = 2
_NUM_SUBCORES = 16
_NUM_WORKERS = _NUM_CORES * _NUM_SUBCORES

_BLK = 128
_NBUF = 4


@functools.lru_cache(maxsize=None)
def _make_gather(n_b: int, n_s: int, d: int):
    assert d == 64 and n_b % _BLK == 0
    n_bh = n_b // _BLK
    n_blocks = n_s * n_bh
    blocks_per_w = n_blocks // _NUM_WORKERS
    assert blocks_per_w % _NBUF == 0 and blocks_per_w >= 2 * _NBUF

    mesh = plsc.VectorSubcoreMesh(
        core_axis_name="c", subcore_axis_name="s",
        num_cores=_NUM_CORES, num_subcores=_NUM_SUBCORES)

    @functools.partial(
        pl.kernel,
        out_type=jax.ShapeDtypeStruct((n_s, 8, n_bh, 8, _BLK), jnp.float32),
        mesh=mesh,
        scratch_types=[
            pltpu.VMEM((blocks_per_w, _BLK), jnp.int32),
            [pltpu.VMEM((_BLK, d), jnp.float32)] * _NBUF,
            [pltpu.VMEM((8, 8, _BLK + 1), jnp.float32)] * _NBUF,
            [pltpu.SemaphoreType.DMA] * _NBUF,
            [pltpu.SemaphoreType.DMA] * _NBUF,
        ],
        compiler_params=pltpu.CompilerParams(
            use_tc_tiling_on_sc=False, needs_layout_passes=False),
    )
    def gather_kernel(table, idx_hbm, out5, idx_v, bufs, tbufs, gsems, ssems):
        wid = lax.axis_index("s") * _NUM_CORES + lax.axis_index("c")
        blk_base = wid * blocks_per_w
        pltpu.sync_copy(idx_hbm.at[pl.ds(blk_base, blocks_per_w)], idx_v)

        d16 = lax.iota(jnp.int32, 16)
        dh_vecs = [(d16 + 16 * k) >> 3 for k in range(4)]
        dl_vecs = [(d16 + 16 * k) & 7 for k in range(4)]

        def fire_gather(i, b):
            pltpu.async_copy(table.at[idx_v.at[i]], bufs[b], gsems[b])

        def wait_gather(b):
            pltpu.make_async_copy(
                table.at[idx_v.at[0]], bufs[b], gsems[b]).wait()

        def fire_store(i, b):
            blk = blk_base + i
            s = blk >> 5
            bh = blk & (n_bh - 1)
            pltpu.async_copy(tbufs[b].at[:, :, pl.ds(0, _BLK)],
                             out5.at[s, :, bh], ssems[b])

        def wait_store(b):
            pltpu.make_async_copy(
                tbufs[b].at[:, :, pl.ds(0, _BLK)],
                out5.at[0, :, 0], ssems[b]).wait()

        def transpose_block(b):
            buf_r, tbuf_r = bufs[b], tbufs[b]

            @pl.loop(0, _BLK, unroll=4)
            def _(bl):
                blv = jnp.full((16,), bl, jnp.int32)
                for k in range(4):
                    v = buf_r[bl, pl.ds(16 * k, 16)]
                    plsc.store_scatter(
                        tbuf_r, [dh_vecs[k], dl_vecs[k], blv], v)

        def step(i, j, guard_store, guard_gather):
            if guard_store:
                wait_store(j)
            ahead = (j + 2) % _NBUF
            if guard_gather:

                @pl.when(i + 2 < blocks_per_w)
                def _():
                    fire_gather(i + 2, ahead)
            else:
                fire_gather(i + 2, ahead)
            wait_gather(j)
            transpose_block(j)
            fire_store(i, j)

        fire_gather(0, 0)
        fire_gather(1, 1)
        for j in range(_NBUF):
            step(j, j, guard_store=False, guard_gather=False)

        @pl.loop(1, blocks_per_w // _NBUF)
        def _(t):
            for j in range(_NBUF):
                step(_NBUF * t + j, j, guard_store=True, guard_gather=True)

        for j in range(_NBUF):
            wait_store(j)

    return gather_kernel


def kernel(inputs, embedding):
    b, s = inputs.shape
    v, d = embedding.shape
    idx = inputs.T.reshape(-1, _BLK).astype(jnp.int32)
    y = _make_gather(b, s, d)(embedding, idx)
    return y.transpose(2, 4, 0, 1, 3).reshape(b, s, d)


# speedup vs baseline: 2.9928x; 1.2995x over previous
import functools

import jax
import jax.numpy as jnp
from jax import lax
from jax.experimental import pallas as pl
from jax.experimental.pallas import tpu as pltpu
from jax.experimental.pallas import tpu_sc as plsc

_NUM_CORES = 2
_NUM_SUBCORES = 16
_NUM_WORKERS = _NUM_CORES * _NUM_SUBCORES

_BLK = 128
_NBUF = 4


@functools.lru_cache(maxsize=None)
def _make_gather(n_b: int, n_s: int, d: int):
    assert d == 64 and n_b % _BLK == 0
    n_bh = n_b // _BLK
    n_blocks = n_s * n_bh
    blocks_per_w = n_blocks // _NUM_WORKERS
    assert blocks_per_w % _NBUF == 0 and blocks_per_w >= 2 * _NBUF

    mesh = plsc.VectorSubcoreMesh(
        core_axis_name="c", subcore_axis_name="s",
        num_cores=_NUM_CORES, num_subcores=_NUM_SUBCORES)

    @functools.partial(
        pl.kernel,
        out_type=jax.ShapeDtypeStruct((n_s, 8, n_bh, 8, _BLK), jnp.float32),
        mesh=mesh,
        scratch_types=[
            pltpu.VMEM((blocks_per_w, _BLK), jnp.int32),
            [pltpu.VMEM((_BLK, d), jnp.float32)] * _NBUF,
            [pltpu.VMEM((8, 8, _BLK + 1), jnp.float32)] * _NBUF,
            [pltpu.SemaphoreType.DMA] * _NBUF,
            [pltpu.SemaphoreType.DMA] * _NBUF,
        ],
        compiler_params=pltpu.CompilerParams(
            use_tc_tiling_on_sc=False, needs_layout_passes=False),
    )
    def gather_kernel(table, idx_hbm, out5, idx_v, bufs, tbufs, gsems, ssems):
        wid = lax.axis_index("s") * _NUM_CORES + lax.axis_index("c")
        blk_base = wid * blocks_per_w
        pltpu.sync_copy(idx_hbm.at[pl.ds(blk_base, blocks_per_w)], idx_v)

        d16 = lax.iota(jnp.int32, 16)
        dh_vecs = [(d16 + 16 * k) >> 3 for k in range(4)]
        dl_vecs = [(d16 + 16 * k) & 7 for k in range(4)]

        def fire_gather(i, b):
            pltpu.async_copy(table.at[idx_v.at[i]], bufs[b], gsems[b])

        def wait_gather(b):
            pltpu.make_async_copy(
                table.at[idx_v.at[0]], bufs[b], gsems[b]).wait()

        def fire_store(i, b):
            blk = blk_base + i
            s = blk >> 5
            bh = blk & (n_bh - 1)
            pltpu.async_copy(tbufs[b].at[:, :, pl.ds(0, _BLK)],
                             out5.at[s, :, bh], ssems[b])

        def wait_store(b):
            pltpu.make_async_copy(
                tbufs[b].at[:, :, pl.ds(0, _BLK)],
                out5.at[0, :, 0], ssems[b]).wait()

        def transpose_block(b):
            buf_r, tbuf_r = bufs[b], tbufs[b]

            @pl.loop(0, _BLK, unroll=8)
            def _(bl):
                blv = jnp.full((16,), bl, jnp.int32)
                vs = [buf_r[bl, pl.ds(16 * k, 16)] for k in range(4)]
                for k in range(4):
                    plsc.store_scatter(
                        tbuf_r, [dh_vecs[k], dl_vecs[k], blv], vs[k])

        def step(i, j, guard_store, guard_gather):
            if guard_store:
                wait_store(j)
            ahead = (j + 2) % _NBUF
            if guard_gather:

                @pl.when(i + 2 < blocks_per_w)
                def _():
                    fire_gather(i + 2, ahead)
            else:
                fire_gather(i + 2, ahead)
            wait_gather(j)
            transpose_block(j)
            fire_store(i, j)

        fire_gather(0, 0)
        fire_gather(1, 1)
        for j in range(_NBUF):
            step(j, j, guard_store=False, guard_gather=False)

        @pl.loop(1, blocks_per_w // _NBUF)
        def _(t):
            for j in range(_NBUF):
                step(_NBUF * t + j, j, guard_store=True, guard_gather=True)

        for j in range(_NBUF):
            wait_store(j)

    return gather_kernel


def kernel(inputs, embedding):
    b, s = inputs.shape
    v, d = embedding.shape
    idx = inputs.T.reshape(-1, _BLK).astype(jnp.int32)
    y = _make_gather(b, s, d)(embedding, idx)
    return y.transpose(2, 4, 0, 1, 3).reshape(b, s, d)
